# baseline (speedup 1.0000x reference)
import jax
import jax.numpy as jnp
from jax.experimental import pallas as pl
from jax.experimental.pallas import tpu as pltpu


_ROWS = 128
_COLS = 32768
_BLOCK_COLS = 8192


_BLOCK_ROWS = 64


def _decomp_block(x_ref, season_ref, trend_ref):
    x = x_ref[...]
    row = jax.lax.broadcasted_iota(jnp.int32, x.shape, 0)
    is_row0 = (row + pl.program_id(0) * _BLOCK_ROWS) == 0
    zero = jnp.zeros_like(x)
    season_ref[...] = jnp.where(is_row0, zero, x)
    trend_ref[...] = jnp.where(is_row0, x, zero)


def kernel(x):
    grid = (_ROWS // _BLOCK_ROWS, _COLS // _BLOCK_COLS)
    spec = pl.BlockSpec((_BLOCK_ROWS, _BLOCK_COLS), lambda i, j: (i, j))
    season, trend = pl.pallas_call(
        _decomp_block,
        grid=grid,
        in_specs=[spec],
        out_specs=[spec, spec],
        out_shape=[
            jax.ShapeDtypeStruct((_ROWS, _COLS), x.dtype),
            jax.ShapeDtypeStruct((_ROWS, _COLS), x.dtype),
        ],
        compiler_params=pltpu.CompilerParams(
            dimension_semantics=("parallel", "parallel"),
        ),
    )(x)
    return (season, trend)


# speedup vs baseline: 1.1279x; 1.1279x over previous
import jax
import jax.numpy as jnp
from jax.experimental import pallas as pl
from jax.experimental.pallas import tpu as pltpu


_ROWS = 128
_COLS = 32768
_BLOCK_COLS = 8192


def _decomp_block(x_ref, season_ref, trend_ref):
    x = x_ref[...]
    row = jax.lax.broadcasted_iota(jnp.int32, x.shape, 0)
    is_row0 = row == 0
    zero = jnp.zeros_like(x)
    season_ref[...] = jnp.where(is_row0, zero, x)
    trend_ref[...] = jnp.where(is_row0, x, zero)


def kernel(x):
    grid = (_COLS // _BLOCK_COLS,)
    spec = pl.BlockSpec((_ROWS, _BLOCK_COLS), lambda j: (0, j))
    season, trend = pl.pallas_call(
        _decomp_block,
        grid=grid,
        in_specs=[spec],
        out_specs=[spec, spec],
        out_shape=[
            jax.ShapeDtypeStruct((_ROWS, _COLS), x.dtype),
            jax.ShapeDtypeStruct((_ROWS, _COLS), x.dtype),
        ],
        compiler_params=pltpu.CompilerParams(
            dimension_semantics=("parallel",),
        ),
    )(x)
    return (season, trend)
